# Initial kernel scaffold; baseline (speedup 1.0000x reference)
import functools

import jax
import jax.numpy as jnp
from jax import lax
from jax.experimental import pallas as pl
from jax.experimental.pallas import tpu as pltpu
from jax.experimental.pallas import tpu_sc as plsc

N, H, W, C = 8, 224, 224, 96
NC, NS = 2, 16
NW = NC * NS
ROWS_PER_N = H // NW
HALF = W // 2
NCHUNK = HALF // 16
CBLK = C // 16

_SCALE = jnp.float32(2.0 / (W - 1))


def _floor_clip(x):
    t = x.astype(jnp.int32)
    tf = t.astype(jnp.float32)
    adj = (tf > x).astype(jnp.int32)
    fl = t - adj
    i0 = jnp.minimum(jnp.maximum(fl, 0), W - 1)
    i1 = jnp.minimum(jnp.maximum(fl + 1, 0), W - 1)
    return i0, i1, i0.astype(jnp.float32), i1.astype(jnp.float32)


def _body(tbl, theta_hbm, out_hbm,
          theta_v, ia_v, ib_v, ic_v, id_v, u_v, up_v, v_v, vp_v,
          bufa, bufb, bufc, bufd, out_v, sem):
    wid = lax.axis_index("s") * NC + lax.axis_index("c")

    pltpu.sync_copy(theta_hbm, theta_v)

    iota = lax.iota(jnp.int32, 16)

    def sample_body(n, _):
        base_row = n * (H * W)
        a00 = theta_v[n * 6 + 0]
        a01 = theta_v[n * 6 + 1]
        a02 = theta_v[n * 6 + 2]
        a10 = theta_v[n * 6 + 3]
        a11 = theta_v[n * 6 + 4]
        a12 = theta_v[n * 6 + 5]

        def half_body(t, _):
            i = wid * ROWS_PER_N + (t >> 1)
            jb = (t & 1) * HALF
            yt = i.astype(jnp.float32) * _SCALE - 1.0

            for k in range(NCHUNK):
                jv = jb + k * 16 + iota
                xt = jv.astype(jnp.float32) * _SCALE - 1.0
                xs = a00 * xt + a01 * yt + a02
                ys = a10 * xt + a11 * yt + a12
                xv = (xs + 1.0) * jnp.float32(W / 2)
                yv = (ys + 1.0) * jnp.float32(H / 2)
                x0, x1, x0f, x1f = _floor_clip(xv)
                y0, y1, y0f, y1f = _floor_clip(yv)
                sl = pl.ds(k * 16, 16)
                ia_v[sl] = base_row + y0 * W + x0
                ib_v[sl] = base_row + y1 * W + x0
                ic_v[sl] = base_row + y0 * W + x1
                id_v[sl] = base_row + y1 * W + x1
                u_v[sl] = x1f - xv
                up_v[sl] = xv - x0f
                v_v[sl] = y1f - yv
                vp_v[sl] = yv - y0f

            ca = pltpu.async_copy(tbl.at[ia_v], bufa, sem)
            cb = pltpu.async_copy(tbl.at[ib_v], bufb, sem)
            cc = pltpu.async_copy(tbl.at[ic_v], bufc, sem)
            cd = pltpu.async_copy(tbl.at[id_v], bufd, sem)
            ca.wait()
            cb.wait()
            cc.wait()
            cd.wait()

            def pix_body(p, _):
                uu = u_v[p]
                uup = up_v[p]
                vv = v_v[p]
                vvp = vp_v[p]
                for c in range(CBLK):
                    cs = pl.ds(c * 16, 16)
                    sa = bufa[p, cs]
                    sb = bufb[p, cs]
                    sc = bufc[p, cs]
                    sd = bufd[p, cs]
                    m1 = vv * sa + vvp * sb
                    m2 = vv * sc + vvp * sd
                    out_v[p, cs] = uu * m1 + uup * m2
                return 0

            lax.fori_loop(0, HALF, pix_body, 0)

            dst = base_row + i * W + jb
            pltpu.sync_copy(out_v, out_hbm.at[pl.ds(dst, HALF)])
            return 0

        lax.fori_loop(0, 2 * ROWS_PER_N, half_body, 0)
        return 0

    lax.fori_loop(0, N, sample_body, 0)


@jax.jit
def _run(tbl, theta_pad):
    mesh = plsc.VectorSubcoreMesh(core_axis_name="c", subcore_axis_name="s")
    f = pl.kernel(
        _body,
        out_type=jax.ShapeDtypeStruct((N * H * W, C), jnp.float32),
        mesh=mesh,
        scratch_types=[
            pltpu.VMEM((64,), jnp.float32),
            pltpu.VMEM((HALF,), jnp.int32),
            pltpu.VMEM((HALF,), jnp.int32),
            pltpu.VMEM((HALF,), jnp.int32),
            pltpu.VMEM((HALF,), jnp.int32),
            pltpu.VMEM((HALF,), jnp.float32),
            pltpu.VMEM((HALF,), jnp.float32),
            pltpu.VMEM((HALF,), jnp.float32),
            pltpu.VMEM((HALF,), jnp.float32),
            pltpu.VMEM((HALF, C), jnp.float32),
            pltpu.VMEM((HALF, C), jnp.float32),
            pltpu.VMEM((HALF, C), jnp.float32),
            pltpu.VMEM((HALF, C), jnp.float32),
            pltpu.VMEM((HALF, C), jnp.float32),
            pltpu.SemaphoreType.DMA,
        ],
    )
    return f(tbl, theta_pad)


def kernel(X, theta):
    tbl = X.reshape(N * H * W, C)
    theta_pad = jnp.concatenate(
        [theta.reshape(-1), jnp.zeros(16, jnp.float32)])
    out = _run(tbl, theta_pad)
    return out.reshape(N, H, W, C)


# speedup vs baseline: 1.2465x; 1.2465x over previous
import functools

import jax
import jax.numpy as jnp
from jax import lax
from jax.experimental import pallas as pl
from jax.experimental.pallas import tpu as pltpu
from jax.experimental.pallas import tpu_sc as plsc

N, H, W, C = 8, 224, 224, 96
NC, NS = 2, 16
NW = NC * NS
ROWS_PER_N = H // NW
HALF = W // 2
NCHUNK = HALF // 16
CBLK = C // 16

_SCALE = 2.0 / (W - 1)


def _bf16_round(x):
    u = lax.bitcast_convert_type(x, jnp.int32)
    lsb = lax.shift_right_logical(u, 16) & 1
    r = (u + 32767 + lsb) & jnp.int32(-65536)
    return lax.bitcast_convert_type(r, jnp.float32)


def _floor_clip(x):
    t = x.astype(jnp.int32)
    tf = t.astype(jnp.float32)
    fl = jnp.where(tf > x, t - 1, t)
    i0 = jnp.minimum(jnp.maximum(fl, 0), W - 1)
    i1 = jnp.minimum(jnp.maximum(fl + 1, 0), W - 1)
    return i0, i1, i0.astype(jnp.float32), i1.astype(jnp.float32)


def _body(tbl, theta_hbm, out_hbm,
          theta_v, ia_v, ib_v, ic_v, id_v, u_v, up_v, v_v, vp_v,
          bufa, bufb, bufc, bufd, out_v, sem):
    wid = lax.axis_index("s") * NC + lax.axis_index("c")

    pltpu.sync_copy(theta_hbm, theta_v)

    iota = lax.iota(jnp.int32, 16)

    def sample_body(n, _):
        base_row = n * (H * W)
        th = _bf16_round(theta_v[pl.ds(n * 6, 16)])
        a00 = th[0]
        a01 = th[1]
        a02 = th[2]
        a10 = th[3]
        a11 = th[4]
        a12 = th[5]

        def half_body(t, _):
            i = wid * ROWS_PER_N + (t >> 1)
            jb = (t & 1) * HALF
            yt = _bf16_round(
                (iota * 0 + i).astype(jnp.float32) * _SCALE - 1.0)

            for k in range(NCHUNK):
                sl = pl.ds(k * 16, 16)
                jv = jb + k * 16 + iota
                xt = _bf16_round(jv.astype(jnp.float32) * _SCALE - 1.0)
                xs = a00 * xt + a01 * yt + a02
                ys = a10 * xt + a11 * yt + a12
                xv = (xs + 1.0) * (W / 2)
                yv = (ys + 1.0) * (H / 2)
                x0, x1, x0f, x1f = _floor_clip(xv)
                y0, y1, y0f, y1f = _floor_clip(yv)
                ia_v[sl] = base_row + y0 * W + x0
                ib_v[sl] = base_row + y1 * W + x0
                ic_v[sl] = base_row + y0 * W + x1
                id_v[sl] = base_row + y1 * W + x1
                u_v[sl] = x1f - xv
                up_v[sl] = xv - x0f
                v_v[sl] = y1f - yv
                vp_v[sl] = yv - y0f

            ca = pltpu.async_copy(tbl.at[ia_v], bufa, sem)
            cb = pltpu.async_copy(tbl.at[ib_v], bufb, sem)
            cc = pltpu.async_copy(tbl.at[ic_v], bufc, sem)
            cd = pltpu.async_copy(tbl.at[id_v], bufd, sem)
            ca.wait()
            cb.wait()
            cc.wait()
            cd.wait()

            def pix_chunk(q, _):
                pb = q * 16
                uu16 = u_v[pl.ds(pb, 16)]
                uup16 = up_v[pl.ds(pb, 16)]
                vv16 = v_v[pl.ds(pb, 16)]
                vvp16 = vp_v[pl.ds(pb, 16)]
                for l in range(16):
                    p = pb + l
                    uu = uu16[l]
                    uup = uup16[l]
                    vv = vv16[l]
                    vvp = vvp16[l]
                    for c in range(CBLK):
                        cs = pl.ds(c * 16, 16)
                        sa = bufa[p, cs]
                        sb = bufb[p, cs]
                        sc = bufc[p, cs]
                        sd = bufd[p, cs]
                        m1 = vv * sa + vvp * sb
                        m2 = vv * sc + vvp * sd
                        out_v[p, cs] = uu * m1 + uup * m2
                return 0

            lax.fori_loop(0, NCHUNK, pix_chunk, 0)

            dst = base_row + i * W + jb
            pltpu.sync_copy(out_v, out_hbm.at[pl.ds(dst, HALF)])
            return 0

        lax.fori_loop(0, 2 * ROWS_PER_N, half_body, 0)
        return 0

    lax.fori_loop(0, N, sample_body, 0)


@jax.jit
def _run(tbl, theta_pad):
    mesh = plsc.VectorSubcoreMesh(core_axis_name="c", subcore_axis_name="s")
    f = pl.kernel(
        _body,
        out_type=jax.ShapeDtypeStruct((N * H * W, C), jnp.float32),
        mesh=mesh,
        compiler_params=pltpu.CompilerParams(use_tc_tiling_on_sc=False),
        scratch_types=[
            pltpu.VMEM((64,), jnp.float32),
            pltpu.VMEM((HALF,), jnp.int32),
            pltpu.VMEM((HALF,), jnp.int32),
            pltpu.VMEM((HALF,), jnp.int32),
            pltpu.VMEM((HALF,), jnp.int32),
            pltpu.VMEM((HALF,), jnp.float32),
            pltpu.VMEM((HALF,), jnp.float32),
            pltpu.VMEM((HALF,), jnp.float32),
            pltpu.VMEM((HALF,), jnp.float32),
            pltpu.VMEM((HALF, C), jnp.float32),
            pltpu.VMEM((HALF, C), jnp.float32),
            pltpu.VMEM((HALF, C), jnp.float32),
            pltpu.VMEM((HALF, C), jnp.float32),
            pltpu.VMEM((HALF, C), jnp.float32),
            pltpu.SemaphoreType.DMA,
        ],
    )
    return f(tbl, theta_pad)


def kernel(X, theta):
    tbl = X.reshape(N * H * W, C)
    theta_pad = jnp.concatenate(
        [theta.reshape(-1), jnp.zeros(16, jnp.float32)])
    out = _run(tbl, theta_pad)
    return out.reshape(N, H, W, C)


# speedup vs baseline: 1.3848x; 1.1109x over previous
import functools

import jax
import jax.numpy as jnp
from jax import lax
from jax.experimental import pallas as pl
from jax.experimental.pallas import tpu as pltpu
from jax.experimental.pallas import tpu_sc as plsc

N, H, W, C = 8, 224, 224, 96
NC, NS = 2, 16
NW = NC * NS
ROWS_PER_N = H // NW
HALF = W // 2
NCHUNK = HALF // 16
CBLK = C // 16
CP = 128

_SCALE = 2.0 / (W - 1)


def _bf16_round(x):
    u = lax.bitcast_convert_type(x, jnp.int32)
    lsb = lax.shift_right_logical(u, 16) & 1
    r = (u + 32767 + lsb) & jnp.int32(-65536)
    return lax.bitcast_convert_type(r, jnp.float32)


def _floor_clip(x):
    t = x.astype(jnp.int32)
    tf = t.astype(jnp.float32)
    fl = jnp.where(tf > x, t - 1, t)
    i0 = jnp.minimum(jnp.maximum(fl, 0), W - 1)
    i1 = jnp.minimum(jnp.maximum(fl + 1, 0), W - 1)
    return i0, i1, i0.astype(jnp.float32), i1.astype(jnp.float32)


def _body(tbl, theta_hbm, out_hbm,
          theta_v, ia_v, ib_v, ic_v, id_v, u_v, up_v, v_v, vp_v,
          bufa, bufb, bufc, bufd, out_v, sem):
    wid = lax.axis_index("s") * NC + lax.axis_index("c")

    pltpu.sync_copy(theta_hbm, theta_v)

    iota = lax.iota(jnp.int32, 16)

    def sample_body(n, _):
        base_row = n * (H * W)
        th = _bf16_round(theta_v[pl.ds(n * 6, 16)])
        a00 = th[0]
        a01 = th[1]
        a02 = th[2]
        a10 = th[3]
        a11 = th[4]
        a12 = th[5]

        def half_body(t, _):
            i = wid * ROWS_PER_N + (t >> 1)
            jb = (t & 1) * HALF
            yt = _bf16_round(
                (iota * 0 + i).astype(jnp.float32) * _SCALE - 1.0)

            for k in range(NCHUNK):
                sl = pl.ds(k * 16, 16)
                jv = jb + k * 16 + iota
                xt = _bf16_round(jv.astype(jnp.float32) * _SCALE - 1.0)
                xs = a00 * xt + a01 * yt + a02
                ys = a10 * xt + a11 * yt + a12
                xv = (xs + 1.0) * (W / 2)
                yv = (ys + 1.0) * (H / 2)
                x0, x1, x0f, x1f = _floor_clip(xv)
                y0, y1, y0f, y1f = _floor_clip(yv)
                ia_v[sl] = base_row + y0 * W + x0
                ib_v[sl] = base_row + y1 * W + x0
                ic_v[sl] = base_row + y0 * W + x1
                id_v[sl] = base_row + y1 * W + x1
                u_v[sl] = x1f - xv
                up_v[sl] = xv - x0f
                v_v[sl] = y1f - yv
                vp_v[sl] = yv - y0f

            ca = pltpu.async_copy(tbl.at[ia_v], bufa, sem)
            cb = pltpu.async_copy(tbl.at[ib_v], bufb, sem)
            cc = pltpu.async_copy(tbl.at[ic_v], bufc, sem)
            cd = pltpu.async_copy(tbl.at[id_v], bufd, sem)
            ca.wait()
            cb.wait()
            cc.wait()
            cd.wait()

            def pix_chunk(q, _):
                pb = q * 16
                uu16 = u_v[pl.ds(pb, 16)]
                uup16 = up_v[pl.ds(pb, 16)]
                vv16 = v_v[pl.ds(pb, 16)]
                vvp16 = vp_v[pl.ds(pb, 16)]
                for l in range(16):
                    p = pb + l
                    uu = uu16[l]
                    uup = uup16[l]
                    vv = vv16[l]
                    vvp = vvp16[l]
                    for c in range(CBLK):
                        cs = pl.ds(c * 16, 16)
                        sa = bufa[p, cs]
                        sb = bufb[p, cs]
                        sc = bufc[p, cs]
                        sd = bufd[p, cs]
                        m1 = vv * sa + vvp * sb
                        m2 = vv * sc + vvp * sd
                        out_v[p, cs] = uu * m1 + uup * m2
                return 0

            lax.fori_loop(0, NCHUNK, pix_chunk, 0)

            dst = base_row + i * W + jb
            pltpu.sync_copy(out_v, out_hbm.at[pl.ds(dst, HALF)])
            return 0

        lax.fori_loop(0, 2 * ROWS_PER_N, half_body, 0)
        return 0

    lax.fori_loop(0, N, sample_body, 0)


@jax.jit
def _run(tbl, theta_pad):
    mesh = plsc.VectorSubcoreMesh(core_axis_name="c", subcore_axis_name="s")
    f = pl.kernel(
        _body,
        out_type=jax.ShapeDtypeStruct((N * H * W, C), jnp.float32),
        mesh=mesh,
        scratch_types=[
            pltpu.VMEM((64,), jnp.float32),
            pltpu.VMEM((HALF,), jnp.int32),
            pltpu.VMEM((HALF,), jnp.int32),
            pltpu.VMEM((HALF,), jnp.int32),
            pltpu.VMEM((HALF,), jnp.int32),
            pltpu.VMEM((HALF,), jnp.float32),
            pltpu.VMEM((HALF,), jnp.float32),
            pltpu.VMEM((HALF,), jnp.float32),
            pltpu.VMEM((HALF,), jnp.float32),
            pltpu.VMEM((HALF, CP), jnp.float32),
            pltpu.VMEM((HALF, CP), jnp.float32),
            pltpu.VMEM((HALF, CP), jnp.float32),
            pltpu.VMEM((HALF, CP), jnp.float32),
            pltpu.VMEM((HALF, C), jnp.float32),
            pltpu.SemaphoreType.DMA,
        ],
    )
    return f(tbl, theta_pad)


def kernel(X, theta):
    tbl = jnp.pad(X.reshape(N * H * W, C), ((0, 0), (0, CP - C)))
    theta_pad = jnp.concatenate(
        [theta.reshape(-1), jnp.zeros(16, jnp.float32)])
    out = _run(tbl, theta_pad)
    return out.reshape(N, H, W, C)
